# baseline (speedup 1.0000x reference)
import jax
import jax.numpy as jnp
from jax.experimental import pallas as pl
from jax.experimental.pallas import tpu as pltpu


def _dot(x, wt):
    return jnp.dot(x, wt, preferred_element_type=jnp.float32)

N = 2048
EMB = 64
NF = 16
GS = 128
T = 4
NV = 512
NQ = 256
RB = 128
NBLK = N // RB
CHUNKS = N // 128


def _fused_kernel(a_ref, e_ref, nf_ref, emb0_ref, idx_ref,
                  wta_ref, wtb_ref, w51_ref,
                  emb_out_ref, q_out_ref, delta_out_ref,
                  a_s, t1_s, t3_s, z1_s, deg_s):
    i = pl.program_id(0)
    rows = pl.ds(i * RB, RB)

    a_blk = a_ref[...]
    a_s[rows, :] = a_blk
    deg_s[rows, :] = jnp.sum(a_blk, axis=1, keepdims=True)

    z1_s[rows, :] = jnp.dot(a_blk, emb0_ref[...],
                            preferred_element_type=jnp.float32)

    x = e_ref[...].reshape(RB, CHUNKS, 128)
    e_sum = jnp.sum(jnp.sum(x, axis=1), axis=1, keepdims=True)
    wta = wta_ref[...]
    t3 = jax.nn.relu(_dot(e_sum, wta[464:465]))
    t3_s[rows, :] = _dot(t3, wta[64:128])
    t1_s[rows, :] = _dot(nf_ref[...], wta[448:464])

    @pl.when(i == NBLK - 1)
    def _():
        deg = deg_s[...]
        t1 = t1_s[...]
        t3f = t3_s[...]
        wta = wta_ref[...]
        w2t = wta[0:64]
        w8t = wta[320:448]
        w9t = wtb_ref[...][:, 0:128]
        row = jax.lax.broadcasted_iota(jnp.int32, (N, 1), 0)
        hi_mask = (row >= NV).astype(jnp.float32)

        def get_state(e):
            s = jnp.sum(e[:NV], axis=0, keepdims=True)
            return _dot(s, w9t)

        emb0 = emb0_ref[...]
        state = get_state(emb0)
        a = a_s[...]
        prev = emb0
        emb = emb0
        for t in range(T):
            prev = emb
            if t == 0:
                z = z1_s[...]
            else:
                z = jnp.dot(a, emb, preferred_element_type=jnp.float32)
            t2 = _dot(z, w2t) / deg
            sterm = _dot(state, w8t)
            t2 = t2 + hi_mask * sterm
            emb = jax.nn.relu((t1 + t2) + t3f)
            state = get_state(emb)

        emb_out_ref[...] = emb
        delta_out_ref[...] = prev - emb

        idx = idx_ref[...].reshape(1, NQ)
        row_i = jax.lax.broadcasted_iota(jnp.int32, (N, NQ), 0)
        onehot_t = (row_i == idx).astype(jnp.float32)
        gathered = jax.lax.dot_general(
            onehot_t, emb, (((0,), (0,)), ((), ())),
            preferred_element_type=jnp.float32)
        b_ = jax.nn.relu(_dot(gathered, wta[256:320]))
        a_ = jax.nn.relu(_dot(state, wta[128:256]))
        aq = jnp.sum(a_ * w51_ref[...])
        q = _dot(b_, wtb_ref[...][:, 128:130]) + aq
        q_out_ref[...] = q.T


def kernel(node_feature_matrix, adjacency_matrix, edge_feature_matrix, current_embedding,
           unassigned_decision_variable_indices, assigned_variable_indices,
           W1, W2, W3, W4, W5_1, W5_2, W6, W7, W8, W9):
    e128 = edge_feature_matrix.reshape(N * CHUNKS, 128)
    wta = jnp.concatenate([W2, W3, W6, W7, W8, W1, W4], axis=1).T
    wtb = jnp.concatenate([W9, W5_2], axis=0).T

    emb_out, q_t, delta = pl.pallas_call(
        _fused_kernel,
        grid=(NBLK,),
        in_specs=[
            pl.BlockSpec((RB, N), lambda i: (i, 0)),
            pl.BlockSpec((RB * CHUNKS, 128), lambda i: (i, 0)),
            pl.BlockSpec((RB, NF), lambda i: (i, 0)),
            pl.BlockSpec((N, EMB), lambda i: (0, 0)),
            pl.BlockSpec((NQ,), lambda i: (0,)),
            pl.BlockSpec((465, EMB), lambda i: (0, 0)),
            pl.BlockSpec((EMB, 130), lambda i: (0, 0)),
            pl.BlockSpec((1, EMB), lambda i: (0, 0)),
        ],
        out_specs=(
            pl.BlockSpec((N, EMB), lambda i: (0, 0)),
            pl.BlockSpec((2, NQ), lambda i: (0, 0)),
            pl.BlockSpec((N, EMB), lambda i: (0, 0)),
        ),
        out_shape=(
            jax.ShapeDtypeStruct((N, EMB), jnp.float32),
            jax.ShapeDtypeStruct((2, NQ), jnp.float32),
            jax.ShapeDtypeStruct((N, EMB), jnp.float32),
        ),
        scratch_shapes=[
            pltpu.VMEM((N, N), jnp.float32),
            pltpu.VMEM((N, EMB), jnp.float32),
            pltpu.VMEM((N, EMB), jnp.float32),
            pltpu.VMEM((N, EMB), jnp.float32),
            pltpu.VMEM((N, 1), jnp.float32),
        ],
        compiler_params=pltpu.CompilerParams(vmem_limit_bytes=50 * 1024 * 1024),
    )(adjacency_matrix, e128, node_feature_matrix, current_embedding,
      unassigned_decision_variable_indices, wta, wtb, W5_1)

    return (emb_out, q_t, delta)


# speedup vs baseline: 1.1103x; 1.1103x over previous
import jax
import jax.numpy as jnp
from jax.experimental import pallas as pl
from jax.experimental.pallas import tpu as pltpu


def _dot(x, wt):
    return jnp.dot(x, wt, preferred_element_type=jnp.float32)

N = 2048
EMB = 64
NF = 16
GS = 128
T = 4
NV = 512
NQ = 256
RB = 256
NBLK = N // RB
CHUNKS = N // 128


def _fused_kernel(a_ref, e_ref, nf_ref, emb0_ref, idx_ref,
                  w1_ref, w2_ref, w3_ref, w4_ref, w51_ref, w52_ref,
                  w6_ref, w7_ref, w8_ref, w9_ref,
                  emb_out_ref, q_out_ref, delta_out_ref,
                  a_s, t1_s, t3_s, z1_s, deg_s):
    i = pl.program_id(0)
    rows = pl.ds(i * RB, RB)

    a_blk = a_ref[...]
    a_s[rows, :] = a_blk
    deg_s[rows, :] = jnp.sum(a_blk, axis=1, keepdims=True)

    z1_s[rows, :] = jnp.dot(a_blk, emb0_ref[...],
                            preferred_element_type=jnp.float32)

    x = e_ref[...].reshape(RB, CHUNKS, 128)
    e_sum = jnp.sum(jnp.sum(x, axis=1), axis=1, keepdims=True)
    t3 = jax.nn.relu(_dot(e_sum, w4_ref[...]))
    t3_s[rows, :] = _dot(t3, w3_ref[...])
    t1_s[rows, :] = _dot(nf_ref[...], w1_ref[...])

    @pl.when(i == NBLK - 1)
    def _():
        deg = deg_s[...]
        t1 = t1_s[...]
        t3f = t3_s[...]
        w2 = w2_ref[...]
        w8 = w8_ref[...]
        w9 = w9_ref[...]
        row = jax.lax.broadcasted_iota(jnp.int32, (N, 1), 0)
        hi_mask = (row >= NV).astype(jnp.float32)

        def get_state(e):
            s = jnp.sum(e[:NV], axis=0, keepdims=True)
            return _dot(s, w9)

        emb0 = emb0_ref[...]
        state = get_state(emb0)
        a = a_s[...]
        prev = emb0
        emb = emb0
        for t in range(T):
            prev = emb
            if t == 0:
                z = z1_s[...]
            else:
                z = jnp.dot(a, emb, preferred_element_type=jnp.float32)
            t2 = _dot(z, w2) / deg
            sterm = _dot(state, w8)
            t2 = t2 + hi_mask * sterm
            emb = jax.nn.relu((t1 + t2) + t3f)
            state = get_state(emb)

        emb_out_ref[...] = emb
        delta_out_ref[...] = prev - emb

        idx = idx_ref[...].reshape(1, NQ)
        row_i = jax.lax.broadcasted_iota(jnp.int32, (N, NQ), 0)
        onehot_t = (row_i == idx).astype(jnp.float32)
        gathered = jax.lax.dot_general(
            onehot_t, emb, (((0,), (0,)), ((), ())),
            preferred_element_type=jnp.float32)
        b_ = jax.nn.relu(_dot(gathered, w7_ref[...]))
        a_ = jax.nn.relu(_dot(state, w6_ref[...]))
        aq = jnp.sum(a_ * w51_ref[...])
        q = _dot(b_, w52_ref[...]) + aq
        q_out_ref[...] = q.T


def kernel(node_feature_matrix, adjacency_matrix, edge_feature_matrix, current_embedding,
           unassigned_decision_variable_indices, assigned_variable_indices,
           W1, W2, W3, W4, W5_1, W5_2, W6, W7, W8, W9):
    e128 = edge_feature_matrix.reshape(N * CHUNKS, 128)

    emb_out, q_t, delta = pl.pallas_call(
        _fused_kernel,
        grid=(NBLK,),
        in_specs=[
            pl.BlockSpec((RB, N), lambda i: (i, 0)),
            pl.BlockSpec((RB * CHUNKS, 128), lambda i: (i, 0)),
            pl.BlockSpec((RB, NF), lambda i: (i, 0)),
            pl.BlockSpec((N, EMB), lambda i: (0, 0)),
            pl.BlockSpec((NQ,), lambda i: (0,)),
            pl.BlockSpec((NF, EMB), lambda i: (0, 0)),
            pl.BlockSpec((EMB, EMB), lambda i: (0, 0)),
            pl.BlockSpec((EMB, EMB), lambda i: (0, 0)),
            pl.BlockSpec((1, EMB), lambda i: (0, 0)),
            pl.BlockSpec((1, EMB), lambda i: (0, 0)),
            pl.BlockSpec((EMB, 2), lambda i: (0, 0)),
            pl.BlockSpec((GS, EMB), lambda i: (0, 0)),
            pl.BlockSpec((EMB, EMB), lambda i: (0, 0)),
            pl.BlockSpec((GS, EMB), lambda i: (0, 0)),
            pl.BlockSpec((EMB, GS), lambda i: (0, 0)),
        ],
        out_specs=(
            pl.BlockSpec((N, EMB), lambda i: (0, 0)),
            pl.BlockSpec((2, NQ), lambda i: (0, 0)),
            pl.BlockSpec((N, EMB), lambda i: (0, 0)),
        ),
        out_shape=(
            jax.ShapeDtypeStruct((N, EMB), jnp.float32),
            jax.ShapeDtypeStruct((2, NQ), jnp.float32),
            jax.ShapeDtypeStruct((N, EMB), jnp.float32),
        ),
        scratch_shapes=[
            pltpu.VMEM((N, N), jnp.float32),
            pltpu.VMEM((N, EMB), jnp.float32),
            pltpu.VMEM((N, EMB), jnp.float32),
            pltpu.VMEM((N, EMB), jnp.float32),
            pltpu.VMEM((N, 1), jnp.float32),
        ],
        compiler_params=pltpu.CompilerParams(vmem_limit_bytes=50 * 1024 * 1024),
    )(adjacency_matrix, e128, node_feature_matrix, current_embedding,
      unassigned_decision_variable_indices,
      W1.T, W2.T, W3.T, W4.T, W5_1, W5_2.T, W6.T, W7.T, W8.T, W9.T)

    return (emb_out, q_t, delta)
